# baseline (speedup 1.0000x reference)
import jax
import jax.numpy as jnp
from jax.experimental import pallas as pl
from jax.experimental.pallas import tpu as pltpu

_N = 16384
_D = 64
_BS = 2048
_BI = 256
_NSPLIT = 4
_BK = _N // _NSPLIT


def _support_body(x_ref, w_ref, s_ref):
    s_ref[...] = jnp.dot(x_ref[...], w_ref[...],
                         preferred_element_type=jnp.float32)


def _gcn_body(*refs):
    adj_refs = refs[:_NSPLIT]
    s_ref, b_ref, o_ref = refs[_NSPLIT:]
    acc = b_ref[...].astype(jnp.float32)
    for j in range(_NSPLIT):
        acc = acc + jnp.dot(adj_refs[j][...],
                            s_ref[pl.ds(j * _BK, _BK), :],
                            preferred_element_type=jnp.float32)
    o_ref[...] = acc


def kernel(input_features, adj, weight, bias):
    support = pl.pallas_call(
        _support_body,
        grid=(_N // _BS,),
        in_specs=[
            pl.BlockSpec((_BS, _D), lambda i: (i, 0)),
            pl.BlockSpec((_D, _D), lambda i: (0, 0)),
        ],
        out_specs=pl.BlockSpec((_BS, _D), lambda i: (i, 0)),
        out_shape=jax.ShapeDtypeStruct((_N, _D), jnp.float32),
        compiler_params=pltpu.CompilerParams(
            dimension_semantics=("arbitrary",)),
    )(input_features, weight)

    adj_specs = [
        pl.BlockSpec((_BI, _BK), lambda i, j=j: (i, j))
        for j in range(_NSPLIT)
    ]
    out = pl.pallas_call(
        _gcn_body,
        grid=(_N // _BI,),
        in_specs=adj_specs + [
            pl.BlockSpec((_N, _D), lambda i: (0, 0)),
            pl.BlockSpec((1, _D), lambda i: (0, 0)),
        ],
        out_specs=pl.BlockSpec((_BI, _D), lambda i: (i, 0)),
        out_shape=jax.ShapeDtypeStruct((_N, _D), jnp.float32),
        compiler_params=pltpu.CompilerParams(
            dimension_semantics=("parallel",)),
    )(*([adj] * _NSPLIT), support, bias.reshape(1, _D))
    return out


# speedup vs baseline: 1.0310x; 1.0310x over previous
import jax
import jax.numpy as jnp
from jax.experimental import pallas as pl
from jax.experimental.pallas import tpu as pltpu

_N = 16384
_D = 64
_BS = 2048
_BI = 256
_NSPLIT = 4
_BK = _N // _NSPLIT


def _support_body(x_ref, w_ref, s_ref):
    s_ref[...] = jnp.dot(x_ref[...], w_ref[...],
                         preferred_element_type=jnp.float32)


def _gcn_body(*refs):
    adj_refs = refs[:_NSPLIT]
    s_ref, b_ref, o_ref = refs[_NSPLIT:]
    acc = b_ref[...].astype(jnp.float32)
    for j in range(_NSPLIT):
        acc = acc + jnp.dot(adj_refs[j][...],
                            s_ref[pl.ds(j * _BK, _BK), :],
                            preferred_element_type=jnp.float32)
    o_ref[...] = acc


def kernel(input_features, adj, weight, bias):
    support = jnp.dot(input_features, weight)

    adj_specs = [
        pl.BlockSpec((_BI, _BK), lambda i, j=j: (i, j))
        for j in range(_NSPLIT)
    ]
    out = pl.pallas_call(
        _gcn_body,
        grid=(_N // _BI,),
        in_specs=adj_specs + [
            pl.BlockSpec((_N, _D), lambda i: (0, 0)),
            pl.BlockSpec((1, _D), lambda i: (0, 0)),
        ],
        out_specs=pl.BlockSpec((_BI, _D), lambda i: (i, 0)),
        out_shape=jax.ShapeDtypeStruct((_N, _D), jnp.float32),
        compiler_params=pltpu.CompilerParams(
            dimension_semantics=("parallel",)),
    )(*([adj] * _NSPLIT), support, bias.reshape(1, _D))
    return out
